# baseline (speedup 1.0000x reference)
import functools

import numpy as np
import jax
import jax.numpy as jnp
from jax import lax
from jax.experimental import pallas as pl
from jax.experimental.pallas import tpu as pltpu
from jax.experimental.pallas import tpu_sc as plsc

N_NODE = 10000
N_EDGE = 160000
HID = 256
HALF = 128
NC, NS = 2, 16
EPW = N_EDGE // NS
KB = 80
NB = EPW // KB
NPAD = 10240
RPT = NPAD // NS
GRID = 10
ROWS = N_NODE // GRID


def _pre_body(r_ref, l_ref, wl_ref, bl_ref, wr_ref, a_ref, b_ref):
    a = jnp.dot(r_ref[...], wl_ref[...], preferred_element_type=jnp.float32)
    a = a + bl_ref[...]
    b = jnp.dot(l_ref[...], wr_ref[...], preferred_element_type=jnp.float32)
    a_ref[0] = a[:, :HALF]
    a_ref[1] = a[:, HALF:]
    b_ref[0] = b[:, :HALF]
    b_ref[1] = b[:, HALF:]


def _pre(right, left, wl, bl, wr):
    return pl.pallas_call(
        _pre_body,
        grid=(GRID,),
        in_specs=[
            pl.BlockSpec((ROWS, HID), lambda i: (i, 0)),
            pl.BlockSpec((ROWS, HID), lambda i: (i, 0)),
            pl.BlockSpec((HID, HID), lambda i: (0, 0)),
            pl.BlockSpec((1, HID), lambda i: (0, 0)),
            pl.BlockSpec((HID, HID), lambda i: (0, 0)),
        ],
        out_specs=[
            pl.BlockSpec((2, ROWS, HALF), lambda i: (0, i, 0)),
            pl.BlockSpec((2, ROWS, HALF), lambda i: (0, i, 0)),
        ],
        out_shape=[
            jax.ShapeDtypeStruct((2, N_NODE, HALF), jnp.float32),
            jax.ShapeDtypeStruct((2, N_NODE, HALF), jnp.float32),
        ],
    )(right, left, wl, bl, wr)


def _post_body(s_ref, d_ref, r_ref, wf_ref, bf_ref, wo1_ref, bo1_ref,
               wo2_ref, bo2_ref, o_ref):
    s = jnp.concatenate([s_ref[0], s_ref[1]], axis=1)
    agg = jnp.dot(s, wf_ref[...], preferred_element_type=jnp.float32,
                  precision=lax.Precision.HIGHEST)
    agg = agg + jnp.dot(d_ref[...], bf_ref[...],
                        preferred_element_type=jnp.float32,
                        precision=lax.Precision.HIGHEST)
    zin = jnp.concatenate([agg, r_ref[...]], axis=1)
    z = jnp.dot(zin, wo1_ref[...], preferred_element_type=jnp.float32)
    z = jnp.maximum(z + bo1_ref[...], 0.0)
    o_ref[...] = jnp.dot(z, wo2_ref[...],
                         preferred_element_type=jnp.float32) + bo2_ref[...]


def _post(s, deg16, right, wf, bf16, wo1, bo1, wo2, bo2):
    return pl.pallas_call(
        _post_body,
        grid=(GRID,),
        in_specs=[
            pl.BlockSpec((2, ROWS, HALF), lambda i: (0, i, 0)),
            pl.BlockSpec((ROWS, HALF), lambda i: (i, 0)),
            pl.BlockSpec((ROWS, HID), lambda i: (i, 0)),
            pl.BlockSpec((HID, HID), lambda i: (0, 0)),
            pl.BlockSpec((HALF, HID), lambda i: (0, 0)),
            pl.BlockSpec((2 * HID, HID), lambda i: (0, 0)),
            pl.BlockSpec((1, HID), lambda i: (0, 0)),
            pl.BlockSpec((HID, HID), lambda i: (0, 0)),
            pl.BlockSpec((1, HID), lambda i: (0, 0)),
        ],
        out_specs=pl.BlockSpec((ROWS, HID), lambda i: (i, 0)),
        out_shape=jax.ShapeDtypeStruct((N_NODE, HID), jnp.float32),
    )(s, deg16, right, wf, bf16, wo1, bo1, wo2, bo2)


def _mlp1_body(h_ref, w_ref, b_ref, y_ref, cs_ref, cq_ref):
    y = jnp.dot(h_ref[...], w_ref[...], preferred_element_type=jnp.float32)
    y = y + b_ref[...]
    y_ref[...] = y

    @pl.when(pl.program_id(0) == 0)
    def _():
        cs_ref[...] = jnp.zeros_like(cs_ref)
        cq_ref[...] = jnp.zeros_like(cq_ref)

    cs_ref[...] += jnp.sum(y, axis=0, keepdims=True)
    cq_ref[...] += jnp.sum(y * y, axis=0, keepdims=True)


def _mlp1(h, w, b):
    return pl.pallas_call(
        _mlp1_body,
        grid=(GRID,),
        in_specs=[
            pl.BlockSpec((ROWS, HID), lambda i: (i, 0)),
            pl.BlockSpec((HID, HID), lambda i: (0, 0)),
            pl.BlockSpec((1, HID), lambda i: (0, 0)),
        ],
        out_specs=[
            pl.BlockSpec((ROWS, HID), lambda i: (i, 0)),
            pl.BlockSpec((1, HID), lambda i: (0, 0)),
            pl.BlockSpec((1, HID), lambda i: (0, 0)),
        ],
        out_shape=[
            jax.ShapeDtypeStruct((N_NODE, HID), jnp.float32),
            jax.ShapeDtypeStruct((1, HID), jnp.float32),
            jax.ShapeDtypeStruct((1, HID), jnp.float32),
        ],
    )(h, w, b)


def _mlp2_body(y_ref, sc_ref, sh_ref, w_ref, b_ref, o_ref):
    h = jnp.maximum(y_ref[...] * sc_ref[...] + sh_ref[...], 0.0)
    o_ref[...] = jnp.dot(h, w_ref[...],
                         preferred_element_type=jnp.float32) + b_ref[...]


def _mlp2(y, scale, shift, w, b):
    nout = w.shape[1]
    return pl.pallas_call(
        _mlp2_body,
        grid=(GRID,),
        in_specs=[
            pl.BlockSpec((ROWS, HID), lambda i: (i, 0)),
            pl.BlockSpec((1, HID), lambda i: (0, 0)),
            pl.BlockSpec((1, HID), lambda i: (0, 0)),
            pl.BlockSpec((HID, nout), lambda i: (0, 0)),
            pl.BlockSpec((1, nout), lambda i: (0, 0)),
        ],
        out_specs=pl.BlockSpec((ROWS, nout), lambda i: (i, 0)),
        out_shape=jax.ShapeDtypeStruct((N_NODE, nout), jnp.float32),
    )(y, scale, shift, w, b)


def _head(h, wm, bm, g, be, wf, bf):
    y, cs, cq = _mlp1(h, wm, bm.reshape(1, HID))
    mean = cs / N_NODE
    var = cq / N_NODE - mean * mean
    inv = g.reshape(1, HID) / jnp.sqrt(var + 1e-5)
    shift = be.reshape(1, HID) - mean * inv
    return _mlp2(y, inv, shift, wf, bf.reshape(1, wf.shape[1]))


def _edge_body(acat, bcat, idx_a, idx_b, idx_a2, idx_b2, ef, wz, z128,
               s_out,
               s_sh, ia_v, ia_off_v, ib_v, ef_v, a_v, b_v, w_v,
               sem_a, sem_b):
    c = lax.axis_index("c")
    s = lax.axis_index("s")

    pltpu.sync_copy(z128, s_sh.at[pl.ds(s * RPT, RPT)])
    pltpu.sync_copy(wz.at[c], w_v)
    plsc.subcore_barrier()


    def batch(b, carry):
        base = s * EPW + b * KB
        bsl = pl.ds(base, KB)
        pltpu.sync_copy(idx_a.at[bsl], ia_v)
        pltpu.sync_copy(ef.at[bsl], ef_v)

        @pl.when(c == 0)
        def _():
            pltpu.sync_copy(idx_a.at[bsl], ia_off_v)
            pltpu.sync_copy(idx_b.at[bsl], ib_v)

        @pl.when(c == 1)
        def _():
            pltpu.sync_copy(idx_a2.at[bsl], ia_off_v)
            pltpu.sync_copy(idx_b2.at[bsl], ib_v)

        ga = pltpu.async_copy(acat.at[ia_off_v], a_v, sem_a)
        gb = pltpu.async_copy(bcat.at[ib_v], b_v, sem_b)
        ga.wait()
        gb.wait()

        def grp(g, carry2):
            efv = ef_v[pl.ds(g * 16, 16)]
            for l in range(16):
                i = g * 16 + l
                eb = lax.gather(
                    efv, jnp.full((16, 1), l, jnp.int32),
                    dimension_numbers=lax.GatherDimensionNumbers(
                        offset_dims=(), collapsed_slice_dims=(0,),
                        start_index_map=(0,)),
                    slice_sizes=(1,),
                    mode=lax.GatherScatterMode.PROMISE_IN_BOUNDS)
                for j in range(HALF // 16):
                    sl = pl.ds(j * 16, 16)
                    m = (a_v[i, sl] + eb * w_v[sl]) + b_v[i, sl]
                    m = jnp.maximum(m, 0.0)
                    sp = m * np.float32(65537.0)
                    a_v[i, sl] = sp - (sp - m)
            return carry2

        lax.fori_loop(0, KB // 16, grp, 0)

        pltpu.sync_copy(a_v, s_sh.at[ia_v], add=True)
        return carry

    lax.fori_loop(0, NB, batch, 0)

    plsc.subcore_barrier()

    sl = pl.ds(s * RPT, RPT)
    pltpu.sync_copy(s_sh.at[sl], s_out.at[c, sl])


def _edge(acat, bcat, idx_a, idx_b, ef, wz):
    z128 = jnp.zeros((RPT, HALF), jnp.float32)
    f = pl.kernel(
        _edge_body,
        out_type=jax.ShapeDtypeStruct((2, NPAD, HALF), jnp.float32),
        mesh=plsc.VectorSubcoreMesh(core_axis_name="c", subcore_axis_name="s",
                                    num_cores=NC, num_subcores=NS),
        scratch_types=[
            pltpu.VMEM_SHARED((NPAD, HALF), jnp.float32),
            pltpu.VMEM((KB,), jnp.int32),
            pltpu.VMEM((KB,), jnp.int32),
            pltpu.VMEM((KB,), jnp.int32),
            pltpu.VMEM((KB,), jnp.float32),
            pltpu.VMEM((KB, HALF), jnp.float32),
            pltpu.VMEM((KB, HALF), jnp.float32),
            pltpu.VMEM((HALF,), jnp.float32),
            pltpu.SemaphoreType.DMA,
            pltpu.SemaphoreType.DMA,
        ],
    )
    return f(acat, bcat, idx_a, idx_b, idx_a + N_NODE, idx_b + N_NODE,
             ef, wz, z128)


def _deg_body(idx_h, o128, z128, deg_out, d_sh, ia_v, ones_v, sem):
    c = lax.axis_index("c")
    s = lax.axis_index("s")
    pltpu.sync_copy(z128, d_sh.at[pl.ds(s * RPT, RPT)])
    pltpu.sync_copy(o128, ones_v)
    plsc.subcore_barrier()

    def batch(b, carry):
        base = s * EPW + b * KB
        pltpu.sync_copy(idx_h.at[pl.ds(base, KB)], ia_v)
        pltpu.sync_copy(ones_v, d_sh.at[ia_v], add=True)
        return carry

    lax.fori_loop(0, NB, batch, 0)
    plsc.subcore_barrier()

    @pl.when(c == 0)
    def _():
        sl = pl.ds(s * RPT, RPT)
        pltpu.sync_copy(d_sh.at[sl], deg_out.at[sl])


def _deg(idx):
    z128 = jnp.zeros((RPT, HALF), jnp.float32)
    o128 = jnp.ones((KB, HALF), jnp.float32)
    f = pl.kernel(
        _deg_body,
        out_type=jax.ShapeDtypeStruct((NPAD, HALF), jnp.float32),
        mesh=plsc.VectorSubcoreMesh(core_axis_name="c", subcore_axis_name="s",
                                    num_cores=NC, num_subcores=NS),
        scratch_types=[
            pltpu.VMEM_SHARED((NPAD, HALF), jnp.float32),
            pltpu.VMEM((KB,), jnp.int32),
            pltpu.VMEM((KB, HALF), jnp.float32),
            pltpu.SemaphoreType.DMA,
        ],
    )
    return f(idx, o128, z128)


def _fourier_jnp(x):
    scales = (2.0 ** np.arange(-16, 16)).astype(np.float32)
    srow = jnp.asarray(np.repeat(1.0 / scales, 4)[None, :])
    ms = jnp.tile(x, (1, 32)) * srow
    return jnp.concatenate([jnp.sin(ms), jnp.cos(ms)], axis=1)


def _conv(p, left, src_i, dst_i, ef, deg128, right):
    s = p['scale_final'][0]
    sp = p['scale_post'][0]
    acat, bcat = _pre(right, left, p['W_left'] * s,
                      (p['b_left'] * s).reshape(1, HID), p['W_right'] * s)
    wz = (p['W_edge'][0] * s).reshape(2, HALF)
    s_acc = _edge(acat.reshape(2 * N_NODE, HALF),
                  bcat.reshape(2 * N_NODE, HALF),
                  dst_i, src_i, ef, wz)
    bf128 = jnp.tile(((p['b_final'] * sp) / HALF)[None, :], (HALF, 1))
    wf = p['W_final'].astype(jnp.bfloat16).astype(jnp.float32) * sp
    return _post(s_acc, deg128, right, wf, bf128,
                 p['W_o1'], p['b_o1'].reshape(1, HID),
                 p['W_o2'], p['b_o2'].reshape(1, HID))


def kernel(xs, edge_indices, edge_features, xt, params):
    src = edge_indices[0]
    dst = edge_indices[1]
    ef = edge_features[:, 0]
    hxs = _fourier_jnp(xs)
    hxt = _fourier_jnp(xt)
    deg_dst = _deg(dst)
    deg_src = _deg(src)
    for l in range(2):
        hxt = _conv(params['conv_s_t'][l], hxs, src, dst, ef, deg_dst, hxt)
        hxs = _conv(params['conv_t_s'][l], hxt, dst, src, ef, deg_src, hxs)
    xs_out = _head(hxs, params['W_mhs'], params['b_mhs'], params['g_bns'],
                   params['be_bns'], params['W_fhs'], params['b_fhs'])
    xt_out = _head(hxt, params['W_mht'], params['b_mht'], params['g_bnt'],
                   params['be_bnt'], params['W_fht'], params['b_fht'])
    return xs_out, xt_out
